# Initial kernel scaffold; baseline (speedup 1.0000x reference)
import jax
import jax.numpy as jnp
from jax.experimental import pallas as pl


def kernel(x, edge_index, W1, a_src1, a_dst1, b1, W2, a_src2, a_dst2, b2, W3, a_src3, a_dst3, b3):
    raise NotImplementedError("write your pallas kernel here")


# speedup vs baseline: 44.3386x; 44.3386x over previous
import functools

import jax
import jax.numpy as jnp
from jax import lax
from jax.experimental import pallas as pl
from jax.experimental.pallas import tpu as pltpu
from jax.experimental.pallas import tpu_sc as plsc

_GDN = lax.GatherDimensionNumbers(offset_dims=(), collapsed_slice_dims=(0,),
                                  start_index_map=(0,))

N = 10000
NP = 10240
DH = 128
HID = 16
HEADS = 8
D_OUT = 16
E = 320000
E_TOT = E + N
NW = 32
C = 40
CHUNKS_PER_W = (E_TOT + NW * C - 1) // (NW * C)
E_PAD = NW * CHUNKS_PER_W * C
ROWS_PER_SUB = NP // 16
BLK = 256
EPS = 1e-16


def _pack_tables_wide(h, a_s, a_d):
    r = lax.broadcasted_iota(jnp.int32, (DH, DH), 0)
    c = lax.broadcasted_iota(jnp.int32, (DH, DH), 1)
    ms = jnp.where((c < 16) & (r // HID == c % HEADS), 1.0, 0.0)
    md = jnp.where((c >= 16) & (c < 32) & (r // HID == (c - 16) % HEADS),
                   1.0, 0.0)
    return (jnp.dot(h * a_s, ms.astype(jnp.float32),
                    preferred_element_type=jnp.float32) +
            jnp.dot(h * a_d, md.astype(jnp.float32),
                    preferred_element_type=jnp.float32))


def _pack_tables_final(h, a_s, a_d):
    r = lax.broadcasted_iota(jnp.int32, (D_OUT, DH), 0)
    c = lax.broadcasted_iota(jnp.int32, (D_OUT, DH), 1)
    mh = jnp.where((c < 16) & (r == c), 1.0, 0.0)
    ms = jnp.where((c >= 16) & (c < 32), 1.0, 0.0)
    md = jnp.where((c >= 32) & (c < 48), 1.0, 0.0)
    return (jnp.dot(h, mh.astype(jnp.float32),
                    preferred_element_type=jnp.float32) +
            jnp.dot(h * a_s, ms.astype(jnp.float32),
                    preferred_element_type=jnp.float32) +
            jnp.dot(h * a_d, md.astype(jnp.float32),
                    preferred_element_type=jnp.float32))


def _tc_first_body(x_ref, w_ref, as_ref, ad_ref, h_out, t_out):
    h = jnp.dot(x_ref[...], w_ref[...], preferred_element_type=jnp.float32)
    h_out[...] = h
    t_out[...] = _pack_tables_wide(h, as_ref[...], ad_ref[...])


def _merge_norm_elu(o0, o1, s0, s1, b):
    ssum = s0 + s1
    rinv = 1.0 / (ssum[:, :HEADS] + EPS)
    r = lax.broadcasted_iota(jnp.int32, (HEADS, DH), 0)
    c = lax.broadcasted_iota(jnp.int32, (HEADS, DH), 1) // HID
    rb = jnp.where(r == c, 1.0, 0.0).astype(jnp.float32)
    rf = jnp.dot(rinv, rb, preferred_element_type=jnp.float32)
    o = (o0 + o1) * rf + b
    return jnp.where(o > 0.0, o, jnp.exp(jnp.minimum(o, 0.0)) - 1.0)


def _tc_mid_body(o0_ref, o1_ref, s0_ref, s1_ref, b_ref, w_ref, as_ref, ad_ref,
                 h_out, t_out):
    x = _merge_norm_elu(o0_ref[...], o1_ref[...], s0_ref[...], s1_ref[...],
                        b_ref[...])
    h = jnp.dot(x, w_ref[...], preferred_element_type=jnp.float32)
    h_out[...] = h
    t_out[...] = _pack_tables_wide(h, as_ref[...], ad_ref[...])


def _tc_mid3_body(o0_ref, o1_ref, s0_ref, s1_ref, b_ref, w_ref, as_ref,
                  ad_ref, t_out):
    x = _merge_norm_elu(o0_ref[...], o1_ref[...], s0_ref[...], s1_ref[...],
                        b_ref[...])
    h = jnp.dot(x, w_ref[...], preferred_element_type=jnp.float32)
    t_out[...] = _pack_tables_final(h, as_ref[...], ad_ref[...])


def _tc_final_body(o0_ref, o1_ref, s0_ref, s1_ref, b_ref, out_ref):
    ssum = s0_ref[...] + s1_ref[...]
    out_ref[...] = (o0_ref[...] + o1_ref[...]) / (ssum + EPS) + b_ref[...]


def _row_spec(width):
    return pl.BlockSpec((BLK, width), lambda i: (i, 0))


def _full_spec(r, w):
    return pl.BlockSpec((r, w), lambda i: (0, 0))


_GRID = (NP // BLK,)


def _tc_first(xp, W, a_s, a_d):
    return pl.pallas_call(
        _tc_first_body,
        grid=_GRID,
        in_specs=[_row_spec(DH), _full_spec(DH, DH), _full_spec(1, DH),
                  _full_spec(1, DH)],
        out_specs=[_row_spec(DH), _row_spec(DH)],
        out_shape=[jax.ShapeDtypeStruct((NP, DH), jnp.float32),
                   jax.ShapeDtypeStruct((NP, DH), jnp.float32)],
    )(xp, W, a_s, a_d)


def _tc_mid(o0, o1, s0, s1, b, W, a_s, a_d):
    return pl.pallas_call(
        _tc_mid_body,
        grid=_GRID,
        in_specs=[_row_spec(DH), _row_spec(DH), _row_spec(16), _row_spec(16),
                  _full_spec(1, DH), _full_spec(DH, DH),
                  _full_spec(1, DH), _full_spec(1, DH)],
        out_specs=[_row_spec(DH), _row_spec(DH)],
        out_shape=[jax.ShapeDtypeStruct((NP, DH), jnp.float32),
                   jax.ShapeDtypeStruct((NP, DH), jnp.float32)],
    )(o0, o1, s0, s1, b, W, a_s, a_d)


def _tc_mid3(o0, o1, s0, s1, b, W, a_s, a_d):
    return pl.pallas_call(
        _tc_mid3_body,
        grid=_GRID,
        in_specs=[_row_spec(DH), _row_spec(DH), _row_spec(16), _row_spec(16),
                  _full_spec(1, DH), _full_spec(DH, D_OUT),
                  _full_spec(1, D_OUT), _full_spec(1, D_OUT)],
        out_specs=_row_spec(DH),
        out_shape=jax.ShapeDtypeStruct((NP, DH), jnp.float32),
    )(o0, o1, s0, s1, b, W, a_s, a_d)


def _tc_final(o0, o1, s0, s1, b):
    return pl.pallas_call(
        _tc_final_body,
        grid=_GRID,
        in_specs=[_row_spec(16), _row_spec(16), _row_spec(16), _row_spec(16),
                  _full_spec(1, 16)],
        out_specs=_row_spec(16),
        out_shape=jax.ShapeDtypeStruct((NP, 16), jnp.float32),
    )(o0, o1, s0, s1, b)


def _make_sc_edge(dfeat, interpret=False):
    wide = dfeat == DH
    mesh = plsc.VectorSubcoreMesh(core_axis_name="c", subcore_axis_name="s",
                                  num_cores=2, num_subcores=16)

    def body(h_hbm, t_hbm, src_hbm, dst_hbm, out_hbm, sum_hbm,
             outacc, sumacc, sidx, didx, srows, drows, wrows, msgrows,
             hrows, sem1, sem2, sem3):
        cid = lax.axis_index("c")
        sid = lax.axis_index("s")
        wid = cid * 16 + sid
        zero16 = jnp.zeros((16,), jnp.float32)
        arng = jnp.arange(16, dtype=jnp.int32)
        nv = dfeat // 16

        def set_idx(ref, base):
            ref[pl.ds(0, 16)] = arng + base
            ref[pl.ds(16, 16)] = arng + (base + 16)
            ref[pl.ds(24, 16)] = arng + (base + 24)

        @pl.loop(0, C)
        def _(r):
            wrows[r, :] = zero16
            for i in range(nv):
                msgrows[r, pl.ds(16 * i, 16)] = zero16
        for kk in range(ROWS_PER_SUB // C):
            set_idx(sidx, sid * ROWS_PER_SUB + kk * C)
            pltpu.sync_copy(msgrows, outacc.at[sidx])
            pltpu.sync_copy(wrows, sumacc.at[sidx])
        plsc.subcore_barrier()

        ebase = wid * (CHUNKS_PER_W * C)

        @pl.loop(0, CHUNKS_PER_W)
        def _(ck):
            eb = ebase + ck * C
            pltpu.sync_copy(src_hbm.at[pl.ds(eb, C)], sidx)
            pltpu.sync_copy(dst_hbm.at[pl.ds(eb, C)], didx)
            cp1 = pltpu.async_copy(t_hbm.at[sidx], srows, sem1)
            cp2 = pltpu.async_copy(t_hbm.at[didx], drows, sem2)
            if wide:
                cp3 = pltpu.async_copy(h_hbm.at[sidx], hrows, sem3)
            cp1.wait()
            cp2.wait()
            if wide:
                cp3.wait()

            so = 0 if wide else 16
            do = 16 if wide else 32

            @pl.loop(0, C)
            def _(e):
                lv = srows[e, pl.ds(so, 16)] + drows[e, pl.ds(do, 16)]
                lv = jnp.where(lv >= 0.0, lv, 0.2 * lv)
                wv = jnp.exp(lv)
                wrows[e, :] = wv
                if not wide:
                    msgrows[e, :] = srows[e, pl.ds(0, 16)] * wv
                else:
                    for hh in range(HEADS):
                        hv = hrows[e, pl.ds(16 * hh, 16)]
                        wb = lax.gather(
                            wv, jnp.full((16, 1), hh, jnp.int32), _GDN,
                            slice_sizes=(1,),
                            mode=lax.GatherScatterMode.PROMISE_IN_BOUNDS)
                        msgrows[e, pl.ds(16 * hh, 16)] = hv * wb

            pltpu.sync_copy(wrows, sumacc.at[didx], add=True)
            pltpu.sync_copy(msgrows, outacc.at[didx], add=True)

        plsc.subcore_barrier()

        rb = sid * ROWS_PER_SUB
        ob = cid * NP + rb
        for kk in range(ROWS_PER_SUB // C):
            set_idx(sidx, rb + kk * C)
            pltpu.sync_copy(outacc.at[sidx], msgrows)
            pltpu.sync_copy(sumacc.at[sidx], wrows)
            pltpu.sync_copy(msgrows, out_hbm.at[pl.ds(ob + kk * C, C)])
            pltpu.sync_copy(wrows, sum_hbm.at[pl.ds(ob + kk * C, C)])

    out_type = (jax.ShapeDtypeStruct((2 * NP, dfeat), jnp.float32),
                jax.ShapeDtypeStruct((2 * NP, 16), jnp.float32))

    scratch = [
        pltpu.VMEM_SHARED((NP, dfeat), jnp.float32),
        pltpu.VMEM_SHARED((NP, 16), jnp.float32),
        pltpu.VMEM((C,), jnp.int32),
        pltpu.VMEM((C,), jnp.int32),
        pltpu.VMEM((C, DH), jnp.float32),
        pltpu.VMEM((C, DH), jnp.float32),
        pltpu.VMEM((C, 16), jnp.float32),
        pltpu.VMEM((C, dfeat), jnp.float32),
    ]

    if wide:
        @functools.partial(
            pl.kernel, out_type=out_type, mesh=mesh, interpret=interpret,
            scratch_types=scratch + [pltpu.VMEM((C, DH), jnp.float32),
                                     pltpu.SemaphoreType.DMA,
                                     pltpu.SemaphoreType.DMA,
                                     pltpu.SemaphoreType.DMA])
        def k(h_hbm, t_hbm, src_hbm, dst_hbm, out_hbm, sum_hbm,
              outacc, sumacc, sidx, didx, srows, drows, wrows, msgrows,
              hrows, sem1, sem2, sem3):
            body(h_hbm, t_hbm, src_hbm, dst_hbm, out_hbm, sum_hbm,
                 outacc, sumacc, sidx, didx, srows, drows, wrows, msgrows,
                 hrows, sem1, sem2, sem3)
        return k

    @functools.partial(
        pl.kernel, out_type=out_type, mesh=mesh, interpret=interpret,
        scratch_types=scratch + [pltpu.SemaphoreType.DMA,
                                 pltpu.SemaphoreType.DMA])
    def k16(t_hbm, src_hbm, dst_hbm, out_hbm, sum_hbm,
            outacc, sumacc, sidx, didx, srows, drows, wrows, msgrows,
            sem1, sem2):
        body(None, t_hbm, src_hbm, dst_hbm, out_hbm, sum_hbm,
             outacc, sumacc, sidx, didx, srows, drows, wrows, msgrows,
             None, sem1, sem2, None)
    return k16


def kernel(x, edge_index, W1, a_src1, a_dst1, b1, W2, a_src2, a_dst2, b2,
           W3, a_src3, a_dst3, b3):
    loop = jnp.arange(N, dtype=jnp.int32)
    pad = E_PAD - E_TOT
    src = jnp.concatenate([edge_index[0].astype(jnp.int32), loop,
                           jnp.zeros((pad,), jnp.int32)])
    dst = jnp.concatenate([edge_index[1].astype(jnp.int32), loop,
                           jnp.full((pad,), N, jnp.int32)])
    xp = jnp.pad(x, ((0, NP - N), (0, 0)))

    sc128 = _make_sc_edge(DH)
    sc16 = _make_sc_edge(D_OUT)

    h1, t1 = _tc_first(xp, W1, a_src1.reshape(1, DH), a_dst1.reshape(1, DH))
    o1, m1 = sc128(h1, t1, src, dst)
    h2, t2 = _tc_mid(o1[:NP], o1[NP:], m1[:NP], m1[NP:], b1.reshape(1, DH),
                     W2, a_src2.reshape(1, DH), a_dst2.reshape(1, DH))
    o2, m2 = sc128(h2, t2, src, dst)
    t3 = _tc_mid3(o2[:NP], o2[NP:], m2[:NP], m2[NP:], b2.reshape(1, DH), W3,
                  a_src3.reshape(1, D_OUT), a_dst3.reshape(1, D_OUT))
    o3, m3 = sc16(t3, src, dst)
    out = _tc_final(o3[:NP], o3[NP:], m3[:NP], m3[NP:], b3.reshape(1, D_OUT))
    return out[:N]
